# Initial kernel scaffold; baseline (speedup 1.0000x reference)
import jax
import jax.numpy as jnp
from jax.experimental import pallas as pl


def kernel(outputs1, outputs2, outputs3, outputs4, mimic, targets, n_test):
    raise NotImplementedError("write your pallas kernel here")


# speedup vs baseline: 41.5489x; 41.5489x over previous
import functools

import jax
import jax.numpy as jnp
from jax import lax
from jax.experimental import pallas as pl
from jax.experimental.pallas import tpu as pltpu
from jax.experimental.pallas import tpu_sc as plsc

B = 128
N = 8192
T = 2.0
L = 16
NC = 2
NS = 16
NW = NC * NS
SPW = B // NW
NCH = N // L
UNROLL = 8

_NA = 5


def _row_top2(buf, s, lanes):
    neg = jnp.full((L,), -jnp.inf, jnp.float32)

    def body(i, c):
        t1, t2 = c
        for j in range(UNROLL):
            x = buf[s, pl.ds((i * UNROLL + j) * L, L)]
            t2 = jnp.maximum(t2, jnp.minimum(t1, x))
            t1 = jnp.maximum(t1, x)
        return t1, t2

    t1, t2 = lax.fori_loop(0, NCH // UNROLL, body, (neg, neg))
    m1 = jnp.max(t1)
    ffs = plsc.all_reduce_ffs(t1 == jnp.broadcast_to(m1, (L,)))
    m2 = jnp.max(jnp.where(lanes == ffs, t2, t1))
    return m1, m2


def _sc_entry(o1, o2, o3, o4, mi, tg, out_thr, out_max,
              buf_a, buf_b, tgt_v, thr_v, max_v, sem_a, sem_b):
    cid = lax.axis_index("c")
    sid = lax.axis_index("s")
    wid = sid * NC + cid
    base = wid * SPW
    lanes = lax.iota(jnp.int32, L)

    pltpu.sync_copy(tg, tgt_v)

    arrs = [o1, o2, o3, o4, mi]
    bufs = [buf_a, buf_b]
    sems = [sem_a, sem_b]

    mvec = [jnp.zeros((L,), jnp.float32) for _ in range(SPW)]
    gmax = jnp.float32(-jnp.inf)

    desc = pltpu.async_copy(arrs[0].at[pl.ds(base, SPW)], bufs[0], sems[0])
    for a in range(_NA):
        nxt = None
        if a + 1 < _NA:
            nxt = pltpu.async_copy(
                arrs[a + 1].at[pl.ds(base, SPW)], bufs[(a + 1) % 2],
                sems[(a + 1) % 2])
        desc.wait()
        buf = bufs[a % 2]
        for s in range(SPW):
            m1, m2 = _row_top2(buf, s, lanes)
            tcol = plsc.load_gather(tgt_v, [jnp.full((L,), base + s, jnp.int32)])
            tval = jnp.max(
                plsc.load_gather(buf, [jnp.full((L,), s, jnp.int32), tcol]))
            margin = jnp.where(tval == m1, m1 - m2, jnp.float32(0.0))
            mvec[s] = jnp.where(lanes == a, margin, mvec[s])
            if a < 4:
                gmax = jnp.maximum(gmax, m1)
        desc = nxt

    mask = lanes < _NA
    for s in range(SPW):
        v = mvec[s]
        mx = jnp.max(jnp.where(mask, v, -jnp.inf))
        e = jnp.where(mask, jnp.exp((v - mx) * jnp.float32(1.0 / T)),
                      jnp.float32(0.0))
        thr_v[s] = e / jnp.broadcast_to(jnp.sum(e), (L,))

    max_v[0] = jnp.broadcast_to(gmax, (L,))
    pltpu.sync_copy(thr_v, out_thr.at[pl.ds(base, SPW)])
    pltpu.sync_copy(max_v, out_max.at[pl.ds(wid, 1)])


@jax.jit
def _sc_call(o1, o2, o3, o4, mi, tg):
    mesh = plsc.VectorSubcoreMesh(core_axis_name="c", subcore_axis_name="s")
    entry = functools.partial(
        pl.kernel,
        out_type=[
            jax.ShapeDtypeStruct((B, L), jnp.float32),
            jax.ShapeDtypeStruct((NW, L), jnp.float32),
        ],
        mesh=mesh,
        compiler_params=pltpu.CompilerParams(needs_layout_passes=False),
        scratch_types=[
            pltpu.VMEM((SPW, N), jnp.float32),
            pltpu.VMEM((SPW, N), jnp.float32),
            pltpu.VMEM((B,), jnp.int32),
            pltpu.VMEM((SPW, L), jnp.float32),
            pltpu.VMEM((1, L), jnp.float32),
            pltpu.SemaphoreType.DMA,
            pltpu.SemaphoreType.DMA,
        ],
    )(_sc_entry)
    return entry(o1, o2, o3, o4, mi, tg)


def kernel(outputs1, outputs2, outputs3, outputs4, mimic, targets, n_test):
    del n_test
    thr, pmax = _sc_call(outputs1, outputs2, outputs3, outputs4, mimic,
                         targets.astype(jnp.int32))
    return jnp.max(pmax), thr[:, :_NA]


# speedup vs baseline: 43.0585x; 1.0363x over previous
import functools

import jax
import jax.numpy as jnp
from jax import lax
from jax.experimental import pallas as pl
from jax.experimental.pallas import tpu as pltpu
from jax.experimental.pallas import tpu_sc as plsc

B = 128
N = 8192
T = 2.0
L = 16
NC = 2
NS = 16
NW = NC * NS
SPW = B // NW
NCH = N // L
UNROLL = 8

_NA = 5


def _sc_entry(o1, o2, o3, o4, mi, tg, out_thr, out_max,
              buf, tgt_v, marg_v, thr_v, max_v, sem_a, sem_b):
    cid = lax.axis_index("c")
    sid = lax.axis_index("s")
    wid = sid * NC + cid
    base = wid * SPW
    lanes = lax.iota(jnp.int32, L)
    zeros = jnp.zeros((L,), jnp.float32)
    neg = jnp.full((L,), -jnp.inf, jnp.float32)

    pltpu.sync_copy(tg, tgt_v)
    for s in range(SPW):
        marg_v[s] = zeros

    arrs = [o1, o2, o3, o4, mi]
    blk = lambda r: r.at[pl.ds(base, SPW)]
    sems = [sem_a, sem_b]

    pltpu.async_copy(blk(arrs[0]), buf.at[0], sem_a)

    def arr_body(a, gmax):
        slot = lax.rem(a, 2)
        for k in range(_NA - 1):
            @pl.when(a == k)
            def _():
                pltpu.async_copy(blk(arrs[k + 1]), buf.at[(k + 1) % 2],
                                 sems[(k + 1) % 2])
        @pl.when(slot == 0)
        def _():
            pltpu.make_async_copy(blk(arrs[0]), buf.at[0], sem_a).wait()

        @pl.when(slot == 1)
        def _():
            pltpu.make_async_copy(blk(arrs[0]), buf.at[1], sem_b).wait()

        def smp_body(s, gmax):
            def body(i, c):
                t1, t2 = c
                for j in range(UNROLL):
                    x = buf[slot, s, pl.ds((i * UNROLL + j) * L, L)]
                    t2 = jnp.maximum(t2, jnp.minimum(t1, x))
                    t1 = jnp.maximum(t1, x)
                return t1, t2

            t1, t2 = lax.fori_loop(0, NCH // UNROLL, body, (neg, neg))
            m1 = jnp.max(t1)
            ffs = plsc.all_reduce_ffs(t1 == jnp.broadcast_to(m1, (L,)))
            m2 = jnp.max(jnp.where(lanes == ffs, t2, t1))
            tcol = plsc.load_gather(
                tgt_v, [jnp.broadcast_to(base + s, (L,)).astype(jnp.int32)])
            tval = jnp.max(plsc.load_gather(
                buf, [jnp.broadcast_to(slot, (L,)).astype(jnp.int32),
                      jnp.broadcast_to(s, (L,)).astype(jnp.int32), tcol]))
            margin = jnp.where(tval == m1, m1 - m2, jnp.float32(0.0))
            marg_v[s] = jnp.where(lanes == a, margin, marg_v[s])
            return jnp.where(a < 4, jnp.maximum(gmax, m1), gmax)

        return lax.fori_loop(0, SPW, smp_body, gmax)

    gmax = lax.fori_loop(0, _NA, arr_body, jnp.float32(-jnp.inf))

    mask = lanes < _NA

    def soft_body(s, _):
        v = marg_v[s]
        mx = jnp.max(jnp.where(mask, v, -jnp.inf))
        e = jnp.where(mask, jnp.exp((v - mx) * jnp.float32(1.0 / T)), zeros)
        thr_v[s] = e / jnp.broadcast_to(jnp.sum(e), (L,))
        return 0

    lax.fori_loop(0, SPW, soft_body, 0)

    max_v[0] = jnp.broadcast_to(gmax, (L,))
    pltpu.sync_copy(thr_v, out_thr.at[pl.ds(base, SPW)])
    pltpu.sync_copy(max_v, out_max.at[pl.ds(wid, 1)])


@jax.jit
def _sc_call(o1, o2, o3, o4, mi, tg):
    mesh = plsc.VectorSubcoreMesh(core_axis_name="c", subcore_axis_name="s")
    entry = functools.partial(
        pl.kernel,
        out_type=[
            jax.ShapeDtypeStruct((B, L), jnp.float32),
            jax.ShapeDtypeStruct((NW, L), jnp.float32),
        ],
        mesh=mesh,
        compiler_params=pltpu.CompilerParams(needs_layout_passes=False),
        scratch_types=[
            pltpu.VMEM((2, SPW, N), jnp.float32),
            pltpu.VMEM((B,), jnp.int32),
            pltpu.VMEM((SPW, L), jnp.float32),
            pltpu.VMEM((SPW, L), jnp.float32),
            pltpu.VMEM((1, L), jnp.float32),
            pltpu.SemaphoreType.DMA,
            pltpu.SemaphoreType.DMA,
        ],
    )(_sc_entry)
    return entry(o1, o2, o3, o4, mi, tg)


def kernel(outputs1, outputs2, outputs3, outputs4, mimic, targets, n_test):
    del n_test
    thr, pmax = _sc_call(outputs1, outputs2, outputs3, outputs4, mimic,
                         targets.astype(jnp.int32))
    return jnp.max(pmax), thr[:, :_NA]
